# baseline (speedup 1.0000x reference)
import functools

import numpy as np
import jax
import jax.numpy as jnp
from jax import lax
from jax.experimental import pallas as pl
from jax.experimental.pallas import tpu as pltpu
from jax.experimental.pallas import tpu_sc as plsc

_EMBED_DIM = 128
_MAX_LEN = 5000

_NC, _NS, _L = 2, 16, 16
_NW = _NC * _NS
_B, _S, _T, _D = 16, 8, 2048, _EMBED_DIM
_N = _B * _S * _T
_RPW = _N // _NW
_BATCH_ROWS = _S * _T
_C = 128
_NCHUNK = _RPW // _C


def _pos_table_np() -> np.ndarray:
    pos = np.arange(0, _MAX_LEN, dtype=np.float32)[:, None]
    factor = np.exp(
        np.arange(0, _EMBED_DIM, 2, dtype=np.float32)
        * (-np.log(10000.0) / _EMBED_DIM)
    )
    pe = np.zeros((_MAX_LEN, _EMBED_DIM), dtype=np.float32)
    pe[:, 0::2] = np.sin(pos * factor)
    pe[:, 1::2] = np.cos(pos * factor)
    return pe


_TABLE = _pos_table_np()


def _pe_body(x_hbm, ts_hbm, tab_hbm, out_hbm,
             ts_v, idx_v, tab_s, inbuf, gbuf,
             sem_i0, sem_i1, sem_g0, sem_g1, sem_o0, sem_o1, sem_t):
    sem_i = (sem_i0, sem_i1)
    sem_g = (sem_g0, sem_g1)
    sem_o = (sem_o0, sem_o1)

    sid = lax.axis_index("s")
    wid = sid * _NC + lax.axis_index("c")
    base = wid * _RPW
    bstart = (wid // 2) * _BATCH_ROWS
    half = (wid % 2) * _RPW

    pltpu.async_copy(x_hbm.at[pl.ds(base, _C)], inbuf.at[0], sem_i[0])
    pltpu.async_copy(x_hbm.at[pl.ds(base + _C, _C)], inbuf.at[1], sem_i[1])

    @pl.when(sid == 0)
    def _():
        pltpu.async_copy(tab_hbm, tab_s, sem_t)

    pltpu.sync_copy(ts_hbm.at[pl.ds(bstart, _BATCH_ROWS)], ts_v)

    def _min_body(i, m):
        for k in range(8):
            m = jnp.minimum(m, ts_v[pl.ds((i * 8 + k) * _L, _L)])
        return m

    m0 = ts_v[pl.ds(0, _L)]
    m = lax.fori_loop(0, _BATCH_ROWS // (8 * _L), _min_body, m0)
    mn = m[0]
    for i in range(1, _L):
        mn = jnp.minimum(mn, m[i])

    def _idx_body(j, carry):
        for k in range(_C // _L):
            v = ts_v[pl.ds(half + j * _C + k * _L, _L)]
            idx_v[j, pl.ds(k * _L, _L)] = v - mn
        return carry

    lax.fori_loop(0, _NCHUNK, _idx_body, 0)

    @pl.when(sid == 0)
    def _():
        pltpu.make_async_copy(tab_hbm, tab_s, sem_t).wait()
    plsc.subcore_barrier()

    def _start_in(c, b):
        pltpu.async_copy(x_hbm.at[pl.ds(base + c * _C, _C)], inbuf.at[b], sem_i[b])

    def _start_gather(c, b):
        pltpu.async_copy(tab_s.at[idx_v.at[c]], gbuf.at[b], sem_g[b])

    def _start_out(c, b):
        pltpu.async_copy(gbuf.at[b], out_hbm.at[pl.ds(base + c * _C, _C)], sem_o[b])

    def _wait_in(c, b):
        pltpu.make_async_copy(
            x_hbm.at[pl.ds(base + c * _C, _C)], inbuf.at[b], sem_i[b]).wait()

    def _wait_gather(c, b):
        pltpu.make_async_copy(
            tab_s.at[idx_v.at[c]], gbuf.at[b], sem_g[b]).wait()

    def _wait_out(c, b):
        pltpu.make_async_copy(
            gbuf.at[b], out_hbm.at[pl.ds(base + c * _C, _C)], sem_o[b]).wait()

    _start_gather(0, 0)

    @pl.loop(0, _NCHUNK, step=2)
    def _chunk_loop(i):
        for b in range(2):
            c = i + b
            _wait_in(c, b)
            _wait_gather(c, b)

            @pl.loop(0, _C, step=2)
            def _row_loop(r):
                for rr in range(2):
                    for k in range(_D // _L):
                        sl = pl.ds(k * _L, _L)
                        plsc.addupdate(gbuf.at[b, r + rr, sl], inbuf[b, r + rr, sl])

            _start_out(c, b)

            @pl.when(c + 2 < _NCHUNK)
            def _():
                _start_in(c + 2, b)

            @pl.when(c >= 1)
            def _():
                _wait_out(c - 1, 1 - b)

            @pl.when(c + 1 < _NCHUNK)
            def _():
                _start_gather(c + 1, 1 - b)

    _wait_out(_NCHUNK - 1, (_NCHUNK - 1) % 2)


@functools.partial(jax.jit, static_argnums=())
def _pe_call(x, ts, tab):
    mesh = plsc.VectorSubcoreMesh(core_axis_name="c", subcore_axis_name="s")
    f = pl.kernel(
        _pe_body,
        out_type=jax.ShapeDtypeStruct((_N, _D), jnp.float32),
        mesh=mesh,
        scratch_types=[
            pltpu.VMEM((_BATCH_ROWS,), jnp.int32),
            pltpu.VMEM((_NCHUNK, _C), jnp.int32),
            pltpu.VMEM_SHARED((_MAX_LEN, _D), jnp.float32),
            pltpu.VMEM((2, _C, _D), jnp.float32),
            pltpu.VMEM((2, _C, _D), jnp.float32),
            pltpu.SemaphoreType.DMA,
            pltpu.SemaphoreType.DMA,
            pltpu.SemaphoreType.DMA,
            pltpu.SemaphoreType.DMA,
            pltpu.SemaphoreType.DMA,
            pltpu.SemaphoreType.DMA,
            pltpu.SemaphoreType.DMA,
        ],
    )
    return f(x, ts, tab)


def kernel(input_encoded, timesteps):
    x = input_encoded.reshape(_N, _D)
    ts = timesteps.reshape(_N)
    tab = jnp.asarray(_TABLE)
    out = _pe_call(x, ts, tab)
    return out.reshape(input_encoded.shape)


# speedup vs baseline: 1.2469x; 1.2469x over previous
import functools

import numpy as np
import jax
import jax.numpy as jnp
from jax import lax
from jax.experimental import pallas as pl
from jax.experimental.pallas import tpu as pltpu
from jax.experimental.pallas import tpu_sc as plsc

_EMBED_DIM = 128
_MAX_LEN = 5000

_NC, _NS, _L = 2, 16, 16
_NW = _NC * _NS
_B, _S, _T, _D = 16, 8, 2048, _EMBED_DIM
_N = _B * _S * _T
_RPW = _N // _NW
_BATCH_ROWS = _S * _T
_C = 64
_NCHUNK = _RPW // _C
_NBUF = 4


def _pos_table_np() -> np.ndarray:
    pos = np.arange(0, _MAX_LEN, dtype=np.float32)[:, None]
    factor = np.exp(
        np.arange(0, _EMBED_DIM, 2, dtype=np.float32)
        * (-np.log(10000.0) / _EMBED_DIM)
    )
    pe = np.zeros((_MAX_LEN, _EMBED_DIM), dtype=np.float32)
    pe[:, 0::2] = np.sin(pos * factor)
    pe[:, 1::2] = np.cos(pos * factor)
    return pe


_TABLE = _pos_table_np()


def _pe_body(x_hbm, ts_hbm, tab_hbm, out_hbm,
             ts_v, idx_v, tab_s, inbuf, gbuf, sems):
    sem_i = sems[:_NBUF]
    sem_g = sems[_NBUF:2 * _NBUF]
    sem_o = sems[2 * _NBUF:3 * _NBUF]
    sem_t = sems[3 * _NBUF]

    sid = lax.axis_index("s")
    wid = sid * _NC + lax.axis_index("c")
    base = wid * _RPW
    bstart = (wid // 2) * _BATCH_ROWS
    half = (wid % 2) * _RPW

    for b in range(_NBUF):
        pltpu.async_copy(x_hbm.at[pl.ds(base + b * _C, _C)], inbuf.at[b], sem_i[b])

    @pl.when(sid == 0)
    def _():
        pltpu.async_copy(tab_hbm, tab_s, sem_t)

    def _min_body(i, m):
        for k in range(8):
            m = jnp.minimum(m, ts_v[pl.ds((i * 8 + k) * _L, _L)])
        return m

    pltpu.sync_copy(ts_hbm.at[pl.ds(bstart + (_RPW - half), _RPW)], ts_v)
    m0 = ts_v[pl.ds(0, _L)]
    m = lax.fori_loop(0, _RPW // (8 * _L), _min_body, m0)
    pltpu.sync_copy(ts_hbm.at[pl.ds(bstart + half, _RPW)], ts_v)
    m = lax.fori_loop(0, _RPW // (8 * _L), _min_body, m)
    mn = m[0]
    for i in range(1, _L):
        mn = jnp.minimum(mn, m[i])

    def _idx_body(j, carry):
        for k in range(_C // _L):
            v = ts_v[pl.ds(j * _C + k * _L, _L)]
            idx_v[j, pl.ds(k * _L, _L)] = v - mn
        return carry

    lax.fori_loop(0, _NCHUNK, _idx_body, 0)

    @pl.when(sid == 0)
    def _():
        pltpu.make_async_copy(tab_hbm, tab_s, sem_t).wait()
    plsc.subcore_barrier()

    def _start_in(c, b):
        pltpu.async_copy(x_hbm.at[pl.ds(base + c * _C, _C)], inbuf.at[b], sem_i[b])

    def _start_gather(c, b):
        pltpu.async_copy(tab_s.at[idx_v.at[c]], gbuf.at[b], sem_g[b])

    def _start_out(c, b):
        pltpu.async_copy(gbuf.at[b], out_hbm.at[pl.ds(base + c * _C, _C)], sem_o[b])

    def _wait_in(c, b):
        pltpu.make_async_copy(
            x_hbm.at[pl.ds(base + c * _C, _C)], inbuf.at[b], sem_i[b]).wait()

    def _wait_gather(c, b):
        pltpu.make_async_copy(
            tab_s.at[idx_v.at[c]], gbuf.at[b], sem_g[b]).wait()

    def _wait_out(c, b):
        pltpu.make_async_copy(
            gbuf.at[b], out_hbm.at[pl.ds(base + c * _C, _C)], sem_o[b]).wait()

    for b in range(_NBUF - 1):
        _start_gather(b, b)

    @pl.loop(0, _NCHUNK, step=_NBUF)
    def _chunk_loop(i):
        for b in range(_NBUF):
            c = i + b
            _wait_in(c, b)
            _wait_gather(c, b)

            @pl.loop(0, _C, step=2)
            def _row_loop(r):
                for rr in range(2):
                    for k in range(_D // _L):
                        sl = pl.ds(k * _L, _L)
                        plsc.addupdate(gbuf.at[b, r + rr, sl], inbuf[b, r + rr, sl])

            _start_out(c, b)

            @pl.when(c + _NBUF < _NCHUNK)
            def _():
                _start_in(c + _NBUF, b)

            @pl.when(c >= 1)
            def _():
                _wait_out(c - 1, (b - 1) % _NBUF)

            @pl.when(c + _NBUF - 1 < _NCHUNK)
            def _():
                _start_gather(c + _NBUF - 1, (b - 1) % _NBUF)

    _wait_out(_NCHUNK - 1, (_NCHUNK - 1) % _NBUF)


@functools.partial(jax.jit, static_argnums=())
def _pe_call(x, ts, tab):
    mesh = plsc.VectorSubcoreMesh(core_axis_name="c", subcore_axis_name="s")
    f = pl.kernel(
        _pe_body,
        out_type=jax.ShapeDtypeStruct((_N, _D), jnp.float32),
        mesh=mesh,
        scratch_types=[
            pltpu.VMEM((_RPW,), jnp.int32),
            pltpu.VMEM((_NCHUNK, _C), jnp.int32),
            pltpu.VMEM_SHARED((_MAX_LEN, _D), jnp.float32),
            pltpu.VMEM((_NBUF, _C, _D), jnp.float32),
            pltpu.VMEM((_NBUF, _C, _D), jnp.float32),
            [pltpu.SemaphoreType.DMA] * (3 * _NBUF + 1),
        ],
    )
    return f(x, ts, tab)


def kernel(input_encoded, timesteps):
    x = input_encoded.reshape(_N, _D)
    ts = timesteps.reshape(_N)
    tab = jnp.asarray(_TABLE)
    out = _pe_call(x, ts, tab)
    return out.reshape(input_encoded.shape)


# speedup vs baseline: 1.2712x; 1.0195x over previous
import functools

import numpy as np
import jax
import jax.numpy as jnp
from jax import lax
from jax.experimental import pallas as pl
from jax.experimental.pallas import tpu as pltpu
from jax.experimental.pallas import tpu_sc as plsc

_EMBED_DIM = 128
_MAX_LEN = 5000

_NC, _NS, _L = 2, 16, 16
_NW = _NC * _NS
_B, _S, _T, _D = 16, 8, 2048, _EMBED_DIM
_N = _B * _S * _T
_RPW = _N // _NW
_BATCH_ROWS = _S * _T
_C = 32
_NCHUNK = _RPW // _C
_NBUF = 8
_IC = 128
_NIR = _RPW // _IC


def _pos_table_np() -> np.ndarray:
    pos = np.arange(0, _MAX_LEN, dtype=np.float32)[:, None]
    factor = np.exp(
        np.arange(0, _EMBED_DIM, 2, dtype=np.float32)
        * (-np.log(10000.0) / _EMBED_DIM)
    )
    pe = np.zeros((_MAX_LEN, _EMBED_DIM), dtype=np.float32)
    pe[:, 0::2] = np.sin(pos * factor)
    pe[:, 1::2] = np.cos(pos * factor)
    return pe


_TABLE = _pos_table_np()


def _pe_body(x_hbm, ts_hbm, tab_hbm, out_hbm,
             ts_v, idx_v, tab_s, inbuf, gbuf, sems):
    sem_i = sems[:_NBUF]
    sem_g = sems[_NBUF:2 * _NBUF]
    sem_o = sems[2 * _NBUF:3 * _NBUF]
    sem_t = sems[3 * _NBUF]

    sid = lax.axis_index("s")
    wid = sid * _NC + lax.axis_index("c")
    base = wid * _RPW
    bstart = (wid // 2) * _BATCH_ROWS
    half = (wid % 2) * _RPW

    for b in range(_NBUF):
        pltpu.async_copy(x_hbm.at[pl.ds(base + b * _C, _C)], inbuf.at[b], sem_i[b])

    @pl.when(sid == 0)
    def _():
        pltpu.async_copy(tab_hbm, tab_s, sem_t)

    def _min_body(i, m):
        for k in range(8):
            m = jnp.minimum(m, ts_v[pl.ds((i * 8 + k) * _L, _L)])
        return m

    pltpu.sync_copy(ts_hbm.at[pl.ds(bstart + (_RPW - half), _RPW)], ts_v)
    m0 = ts_v[pl.ds(0, _L)]
    m = lax.fori_loop(0, _RPW // (8 * _L), _min_body, m0)
    pltpu.sync_copy(ts_hbm.at[pl.ds(bstart + half, _RPW)], ts_v)
    m = lax.fori_loop(0, _RPW // (8 * _L), _min_body, m)
    mn = m[0]
    for i in range(1, _L):
        mn = jnp.minimum(mn, m[i])

    def _idx_body(j, carry):
        for k in range(_IC // _L):
            v = ts_v[pl.ds(j * _IC + k * _L, _L)]
            idx_v[j, pl.ds(k * _L, _L)] = v - mn
        return carry

    lax.fori_loop(0, _NIR, _idx_body, 0)

    @pl.when(sid == 0)
    def _():
        pltpu.make_async_copy(tab_hbm, tab_s, sem_t).wait()
    plsc.subcore_barrier()

    def _start_in(c, b):
        pltpu.async_copy(x_hbm.at[pl.ds(base + c * _C, _C)], inbuf.at[b], sem_i[b])

    def _idx_ref(c):
        return idx_v.at[(c * _C) // _IC, pl.ds((c * _C) % _IC, _C)]

    def _start_gather(c, b):
        pltpu.async_copy(tab_s.at[_idx_ref(c)], gbuf.at[b], sem_g[b])

    def _start_out(c, b):
        pltpu.async_copy(gbuf.at[b], out_hbm.at[pl.ds(base + c * _C, _C)], sem_o[b])

    def _wait_in(c, b):
        pltpu.make_async_copy(
            x_hbm.at[pl.ds(base + c * _C, _C)], inbuf.at[b], sem_i[b]).wait()

    def _wait_gather(c, b):
        pltpu.make_async_copy(
            tab_s.at[_idx_ref(c)], gbuf.at[b], sem_g[b]).wait()

    def _wait_out(c, b):
        pltpu.make_async_copy(
            gbuf.at[b], out_hbm.at[pl.ds(base + c * _C, _C)], sem_o[b]).wait()

    for b in range(_NBUF - 1):
        _start_gather(b, b)

    @pl.loop(0, _NCHUNK, step=_NBUF)
    def _chunk_loop(i):
        for b in range(_NBUF):
            c = i + b
            _wait_in(c, b)
            _wait_gather(c, b)

            @pl.loop(0, _C, step=2)
            def _row_loop(r):
                for rr in range(2):
                    for k in range(_D // _L):
                        sl = pl.ds(k * _L, _L)
                        plsc.addupdate(gbuf.at[b, r + rr, sl], inbuf[b, r + rr, sl])

            _start_out(c, b)

            @pl.when(c + _NBUF < _NCHUNK)
            def _():
                _start_in(c + _NBUF, b)

            @pl.when(c >= 1)
            def _():
                _wait_out(c - 1, (b - 1) % _NBUF)

            @pl.when(c + _NBUF - 1 < _NCHUNK)
            def _():
                _start_gather(c + _NBUF - 1, (b - 1) % _NBUF)

    _wait_out(_NCHUNK - 1, (_NCHUNK - 1) % _NBUF)


@functools.partial(jax.jit, static_argnums=())
def _pe_call(x, ts, tab):
    mesh = plsc.VectorSubcoreMesh(core_axis_name="c", subcore_axis_name="s")
    f = pl.kernel(
        _pe_body,
        out_type=jax.ShapeDtypeStruct((_N, _D), jnp.float32),
        mesh=mesh,
        scratch_types=[
            pltpu.VMEM((_RPW,), jnp.int32),
            pltpu.VMEM((_NIR, _IC), jnp.int32),
            pltpu.VMEM_SHARED((_MAX_LEN, _D), jnp.float32),
            pltpu.VMEM((_NBUF, _C, _D), jnp.float32),
            pltpu.VMEM((_NBUF, _C, _D), jnp.float32),
            [pltpu.SemaphoreType.DMA] * (3 * _NBUF + 1),
        ],
    )
    return f(x, ts, tab)


def kernel(input_encoded, timesteps):
    x = input_encoded.reshape(_N, _D)
    ts = timesteps.reshape(_N)
    tab = jnp.asarray(_TABLE)
    out = _pe_call(x, ts, tab)
    return out.reshape(input_encoded.shape)


# speedup vs baseline: 1.2801x; 1.0070x over previous
import functools

import numpy as np
import jax
import jax.numpy as jnp
from jax import lax
from jax.experimental import pallas as pl
from jax.experimental.pallas import tpu as pltpu
from jax.experimental.pallas import tpu_sc as plsc

_EMBED_DIM = 128
_MAX_LEN = 5000

_NC, _NS, _L = 2, 16, 16
_NW = _NC * _NS
_B, _S, _T, _D = 16, 8, 2048, _EMBED_DIM
_N = _B * _S * _T
_RPW = _N // _NW
_BATCH_ROWS = _S * _T
_C = 32
_NCHUNK = _RPW // _C
_NBUF = 8
_IC = 128
_NIR = _RPW // _IC


def _pos_table_np() -> np.ndarray:
    pos = np.arange(0, _MAX_LEN, dtype=np.float32)[:, None]
    factor = np.exp(
        np.arange(0, _EMBED_DIM, 2, dtype=np.float32)
        * (-np.log(10000.0) / _EMBED_DIM)
    )
    pe = np.zeros((_MAX_LEN, _EMBED_DIM), dtype=np.float32)
    pe[:, 0::2] = np.sin(pos * factor)
    pe[:, 1::2] = np.cos(pos * factor)
    return pe


def _pos_table_packed() -> np.ndarray:
    import ml_dtypes
    pe = _pos_table_np()
    t = (pe.reshape(_MAX_LEN, _D // 32, 2, 16)
           .transpose(0, 1, 3, 2)
           .reshape(_MAX_LEN, _D))
    u = t.astype(ml_dtypes.bfloat16).view(np.uint16).astype(np.uint32)
    u = u.reshape(_MAX_LEN, _D // 2, 2)
    w = u[..., 0] | (u[..., 1] << 16)
    return w.astype(np.int32, casting="unsafe") if w.dtype != np.int32 else w


_TABLE = _pos_table_packed().view(np.int32)


def _pe_body(x_hbm, ts_hbm, tab_hbm, out_hbm,
             ts_v, idx_v, tab_s, inbuf, gbuf, sems):
    sem_i = sems[:_NBUF]
    sem_g = sems[_NBUF:2 * _NBUF]
    sem_o = sems[2 * _NBUF:3 * _NBUF]
    sem_t = sems[3 * _NBUF]

    sid = lax.axis_index("s")
    wid = sid * _NC + lax.axis_index("c")
    base = wid * _RPW
    bstart = (wid // 2) * _BATCH_ROWS
    half = (wid % 2) * _RPW

    for b in range(_NBUF):
        pltpu.async_copy(x_hbm.at[pl.ds(base + b * _C, _C)], inbuf.at[b], sem_i[b])

    @pl.when(sid == 0)
    def _():
        pltpu.async_copy(tab_hbm, tab_s, sem_t)

    def _min_body(i, m):
        for k in range(8):
            m = jnp.minimum(m, ts_v[pl.ds((i * 8 + k) * _L, _L)])
        return m

    pltpu.sync_copy(ts_hbm.at[pl.ds(bstart + (_RPW - half), _RPW)], ts_v)
    m0 = ts_v[pl.ds(0, _L)]
    m = lax.fori_loop(0, _RPW // (8 * _L), _min_body, m0)
    pltpu.sync_copy(ts_hbm.at[pl.ds(bstart + half, _RPW)], ts_v)
    m = lax.fori_loop(0, _RPW // (8 * _L), _min_body, m)
    mn = m[0]
    for i in range(1, _L):
        mn = jnp.minimum(mn, m[i])

    def _idx_body(j, carry):
        for k in range(_IC // _L):
            v = ts_v[pl.ds(j * _IC + k * _L, _L)]
            idx_v[j, pl.ds(k * _L, _L)] = v - mn
        return carry

    lax.fori_loop(0, _NIR, _idx_body, 0)

    @pl.when(sid == 0)
    def _():
        pltpu.make_async_copy(tab_hbm, tab_s, sem_t).wait()
    plsc.subcore_barrier()

    def _start_in(c, b):
        pltpu.async_copy(x_hbm.at[pl.ds(base + c * _C, _C)], inbuf.at[b], sem_i[b])

    def _idx_ref(c):
        return idx_v.at[(c * _C) // _IC, pl.ds((c * _C) % _IC, _C)]

    def _start_gather(c, b):
        pltpu.async_copy(tab_s.at[_idx_ref(c)], gbuf.at[b], sem_g[b])

    def _start_out(c, b):
        pltpu.async_copy(inbuf.at[b], out_hbm.at[pl.ds(base + c * _C, _C)], sem_o[b])

    def _wait_in(c, b):
        pltpu.make_async_copy(
            x_hbm.at[pl.ds(base + c * _C, _C)], inbuf.at[b], sem_i[b]).wait()

    def _wait_gather(c, b):
        pltpu.make_async_copy(
            tab_s.at[_idx_ref(c)], gbuf.at[b], sem_g[b]).wait()

    def _wait_out(c, b):
        pltpu.make_async_copy(
            inbuf.at[b], out_hbm.at[pl.ds(base + c * _C, _C)], sem_o[b]).wait()

    for b in range(_NBUF):
        _start_gather(b, b)

    @pl.loop(0, _NCHUNK, step=_NBUF)
    def _chunk_loop(i):
        for b in range(_NBUF):
            c = i + b
            _wait_in(c, b)
            _wait_gather(c, b)

            @pl.loop(0, _C, step=2)
            def _row_loop(r):
                for rr in range(2):
                    for k in range(_D // 32):
                        w = gbuf[b, r + rr, pl.ds(k * _L, _L)]
                        a0 = lax.bitcast_convert_type(w << 16, jnp.float32)
                        a1 = lax.bitcast_convert_type(w & jnp.int32(-65536), jnp.float32)
                        plsc.addupdate(
                            inbuf.at[b, r + rr, pl.ds(k * 32, _L)], a0)
                        plsc.addupdate(
                            inbuf.at[b, r + rr, pl.ds(k * 32 + _L, _L)], a1)

            _start_out(c, b)

            @pl.when(c + _NBUF < _NCHUNK)
            def _():
                _start_gather(c + _NBUF, b)

            @pl.when(c >= 1)
            def _():
                _wait_out(c - 1, (b - 1) % _NBUF)

                @pl.when(c - 1 + _NBUF < _NCHUNK)
                def _():
                    _start_in(c - 1 + _NBUF, (b - 1) % _NBUF)

    _wait_out(_NCHUNK - 1, (_NCHUNK - 1) % _NBUF)


@functools.partial(jax.jit, static_argnums=())
def _pe_call(x, ts, tab):
    mesh = plsc.VectorSubcoreMesh(core_axis_name="c", subcore_axis_name="s")
    f = pl.kernel(
        _pe_body,
        out_type=jax.ShapeDtypeStruct((_N, _D), jnp.float32),
        mesh=mesh,
        scratch_types=[
            pltpu.VMEM((_RPW,), jnp.int32),
            pltpu.VMEM((_NIR, _IC), jnp.int32),
            pltpu.VMEM_SHARED((_MAX_LEN, _D // 2), jnp.int32),
            pltpu.VMEM((_NBUF, _C, _D), jnp.float32),
            pltpu.VMEM((_NBUF, _C, _D // 2), jnp.int32),
            [pltpu.SemaphoreType.DMA] * (3 * _NBUF + 1),
        ],
    )
    return f(x, ts, tab)


def kernel(input_encoded, timesteps):
    x = input_encoded.reshape(_N, _D)
    ts = timesteps.reshape(_N)
    tab = jnp.asarray(_TABLE)
    out = _pe_call(x, ts, tab)
    return out.reshape(input_encoded.shape)
